# baseline (speedup 1.0000x reference)
import functools

import jax
import jax.numpy as jnp
from jax import lax
from jax.experimental import pallas as pl
from jax.experimental.pallas import tpu as pltpu
from jax.experimental.pallas import tpu_sc as plsc

N = 10000
E = 320000
F = 128
NL = 8
NCLS = 40
BETA = 0.1
RW = 0.1
MIX = 1.0 - RW - BETA

NC = 2
NS = 16
LANES = 16
N_PAD = 10240
RPT = N_PAD // NS
CHUNK = 128
CPW = 80
GC = 8
E_PW = CPW * CHUNK
E_PAD = NC * NS * E_PW
DEGW = 16

_MESH = dict(core_axis_name="c", subcore_axis_name="s")


def _sc_degree(dst_hbm, deg_out, didx_v, ones_v, stage_v, acc_sp):
    c = lax.axis_index("c")
    s = lax.axis_index("s")
    wid = c * NS + s
    row0 = s * RPT

    def _fill(i, _):
        ones_v[i, :] = jnp.ones((LANES,), jnp.float32)
        return 0
    lax.fori_loop(0, CHUNK, _fill, 0)

    def _zstage(i, _):
        stage_v[i, :] = jnp.zeros((LANES,), jnp.float32)
        return 0
    lax.fori_loop(0, 128, _zstage, 0)

    for k in range(RPT // 128):
        pltpu.sync_copy(stage_v, acc_sp.at[pl.ds(row0 + k * 128, 128)])
    plsc.subcore_barrier()

    pltpu.sync_copy(dst_hbm.at[wid], didx_v)

    def _body(ci, _):
        pltpu.sync_copy(ones_v, acc_sp.at[didx_v.at[ci]], add=True)
        return 0
    lax.fori_loop(0, CPW, _body, 0)
    plsc.subcore_barrier()

    for k in range(RPT // 128):
        pltpu.sync_copy(acc_sp.at[pl.ds(row0 + k * 128, 128)], stage_v)
        pltpu.sync_copy(stage_v, deg_out.at[c, pl.ds(row0 + k * 128, 128)])


def _sc_propagate(hs_hbm, src_hbm, dst_hbm, p_out,
                  sidx_v, didx_v, rows_v, acc_sp, sem0, sem1):
    c = lax.axis_index("c")
    s = lax.axis_index("s")
    wid = c * NS + s
    row0 = s * RPT

    def _zero(i, _):
        for j in range(F // LANES):
            rows_v[0, i, pl.ds(j * LANES, LANES)] = (
                jnp.zeros((LANES,), jnp.float32))
        return 0
    lax.fori_loop(0, CHUNK, _zero, 0)
    for k in range(RPT // CHUNK):
        pltpu.sync_copy(rows_v.at[0],
                        acc_sp.at[pl.ds(row0 + k * CHUNK, CHUNK)])

    pltpu.sync_copy(src_hbm.at[wid], sidx_v)
    plsc.subcore_barrier()

    def _grp(g, _):
        pltpu.sync_copy(dst_hbm.at[wid, pl.ds(g * GC, GC)], didx_v)
        for k2 in range(GC // 2):
            c0 = g * GC + 2 * k2
            d0 = pltpu.async_copy(hs_hbm.at[sidx_v.at[c0]],
                                  rows_v.at[0], sem0)
            d1 = pltpu.async_copy(hs_hbm.at[sidx_v.at[c0 + 1]],
                                  rows_v.at[1], sem1)
            d0.wait()
            pltpu.sync_copy(rows_v.at[0],
                            acc_sp.at[didx_v.at[2 * k2]], add=True)
            d1.wait()
            pltpu.sync_copy(rows_v.at[1],
                            acc_sp.at[didx_v.at[2 * k2 + 1]], add=True)
        return 0
    lax.fori_loop(0, CPW // GC, _grp, 0)
    plsc.subcore_barrier()

    for k in range(RPT // CHUNK):
        pltpu.sync_copy(acc_sp.at[pl.ds(row0 + k * CHUNK, CHUNK)],
                        rows_v.at[0])
        pltpu.sync_copy(rows_v.at[0],
                        p_out.at[c, pl.ds(row0 + k * CHUNK, CHUNK)])


_deg_call = functools.partial(
    pl.kernel, _sc_degree,
    out_type=jax.ShapeDtypeStruct((NC, N_PAD, DEGW), jnp.float32),
    mesh=plsc.VectorSubcoreMesh(**_MESH),
    scratch_types=[
        pltpu.VMEM((CPW, CHUNK), jnp.int32),
        pltpu.VMEM((CHUNK, DEGW), jnp.float32),
        pltpu.VMEM((128, DEGW), jnp.float32),
        pltpu.VMEM_SHARED((N_PAD, DEGW), jnp.float32),
    ],
)()

_prop_call = functools.partial(
    pl.kernel, _sc_propagate,
    out_type=jax.ShapeDtypeStruct((NC, N_PAD, F), jnp.float32),
    mesh=plsc.VectorSubcoreMesh(**_MESH),
    scratch_types=[
        pltpu.VMEM((CPW, CHUNK), jnp.int32),
        pltpu.VMEM((GC, CHUNK), jnp.int32),
        pltpu.VMEM((2, CHUNK, F), jnp.float32),
        pltpu.VMEM_SHARED((N_PAD, F), jnp.float32),
        pltpu.SemaphoreType.DMA,
        pltpu.SemaphoreType.DMA,
    ],
)()


RB = 1024
GRID = N_PAD // RB


def _tc_prelude(x_ref, w_ref, b_ref, d0_ref, d1_ref,
                h_ref, hs_ref, dis_ref, invd_ref):
    deg = d0_ref[...][:, :1] + d1_ref[...][:, :1] + 1.0
    dis = lax.rsqrt(deg)
    h = jnp.dot(x_ref[...], w_ref[...], preferred_element_type=jnp.float32)
    h = jnp.maximum(h + b_ref[...], 0.0)
    h_ref[...] = h
    hs_ref[...] = h * dis
    dis_ref[...] = dis
    invd_ref[...] = 1.0 / deg


def _tc_layer(p0_ref, p1_ref, h_ref, x0_ref, dis_ref, invd_ref, w_ref, bs_ref,
              hn_ref, hsn_ref):
    h = h_ref[...]
    dis = dis_ref[...]
    hi = (p0_ref[...] + p1_ref[...]) * dis + h * invd_ref[...]
    t = MIX * hi + RW * h + BETA * x0_ref[...]
    hn = jnp.dot(t, w_ref[...], preferred_element_type=jnp.float32)
    bs = bs_ref[...]
    hn = jnp.maximum(hn - bs, 0.0) + bs
    hn_ref[...] = hn
    hsn_ref[...] = hn * dis


def _tc_output(h_ref, w_ref, b_ref, o_ref):
    o_ref[...] = jnp.dot(h_ref[...], w_ref[...],
                         preferred_element_type=jnp.float32) + b_ref[...]


def _rowspec(cols):
    return pl.BlockSpec((RB, cols), lambda i: (i, 0))


def _fullspec(r, cols):
    return pl.BlockSpec((r, cols), lambda i: (0, 0))


_prelude_call = pl.pallas_call(
    _tc_prelude,
    grid=(GRID,),
    in_specs=[_rowspec(F), _fullspec(F, F), _fullspec(1, F),
              _rowspec(DEGW), _rowspec(DEGW)],
    out_specs=[_rowspec(F), _rowspec(F), _rowspec(1), _rowspec(1)],
    out_shape=[jax.ShapeDtypeStruct((N_PAD, F), jnp.float32),
               jax.ShapeDtypeStruct((N_PAD, F), jnp.float32),
               jax.ShapeDtypeStruct((N_PAD, 1), jnp.float32),
               jax.ShapeDtypeStruct((N_PAD, 1), jnp.float32)],
)

_layer_call = pl.pallas_call(
    _tc_layer,
    grid=(GRID,),
    in_specs=[_rowspec(F), _rowspec(F), _rowspec(F), _rowspec(F),
              _rowspec(1), _rowspec(1), _fullspec(F, F), _fullspec(1, F)],
    out_specs=[_rowspec(F), _rowspec(F)],
    out_shape=[jax.ShapeDtypeStruct((N_PAD, F), jnp.float32),
               jax.ShapeDtypeStruct((N_PAD, F), jnp.float32)],
)

_output_call = pl.pallas_call(
    _tc_output,
    grid=(GRID,),
    in_specs=[_rowspec(F), _fullspec(F, F), _fullspec(1, F)],
    out_specs=_rowspec(F),
    out_shape=jax.ShapeDtypeStruct((N_PAD, F), jnp.float32),
)


def kernel(x, edge_index, W_in, b_in, W_gcn, srelu_bias, W_out, b_out):
    src = edge_index[0]
    dst = edge_index[1]
    npad = E_PAD - E
    src_p = jnp.concatenate([src, jnp.zeros((npad,), jnp.int32)])
    dst_p = jnp.concatenate(
        [dst, N + (jnp.arange(npad, dtype=jnp.int32) % (N_PAD - N))])
    src_r = src_p.reshape(NC * NS, CPW, CHUNK)
    dst_r = dst_p.reshape(NC * NS, CPW, CHUNK)

    x_p = jnp.pad(x, ((0, N_PAD - N), (0, 0)))
    Wo = jnp.pad(W_out, ((0, 0), (0, F - NCLS)))
    bo = jnp.pad(b_out, (0, F - NCLS))

    deg = _deg_call(dst_r)
    d0 = deg[0]
    d1 = deg[1]

    h, hs, dis, invd = _prelude_call(x_p, W_in, b_in.reshape(1, F), d0, d1)
    x0 = h
    for i in range(NL):
        P = _prop_call(hs, src_r, dst_r)
        h, hs = _layer_call(P[0], P[1], h, x0, dis, invd,
                            W_gcn[i], srelu_bias[i].reshape(1, F))
    out = _output_call(h, Wo, bo.reshape(1, F))
    return out[:N, :NCLS]


# speedup vs baseline: 1.4962x; 1.4962x over previous
import functools

import jax
import jax.numpy as jnp
from jax import lax
from jax.experimental import pallas as pl
from jax.experimental.pallas import tpu as pltpu
from jax.experimental.pallas import tpu_sc as plsc

N = 10000
E = 320000
F = 128
NL = 8
NCLS = 40
BETA = 0.1
RW = 0.1
MIX = 1.0 - RW - BETA

NC = 2
NS = 16
LANES = 16
N_PAD = 10240
RPT = N_PAD // NS
CHUNK = 128
CPW = 79
E_PW = CPW * CHUNK
E_PAD = NC * NS * E_PW
DEGW = 16

_MESH = dict(core_axis_name="c", subcore_axis_name="s")


def _sc_degree(dst_hbm, deg_out, didx_v, ones_v, stage_v, acc_sp):
    c = lax.axis_index("c")
    s = lax.axis_index("s")
    wid = c * NS + s
    row0 = s * RPT

    def _fill(i, _):
        ones_v[i, :] = jnp.ones((LANES,), jnp.float32)
        return 0
    lax.fori_loop(0, CHUNK, _fill, 0)

    def _zstage(i, _):
        stage_v[i, :] = jnp.zeros((LANES,), jnp.float32)
        return 0
    lax.fori_loop(0, 128, _zstage, 0)

    for k in range(RPT // 128):
        pltpu.sync_copy(stage_v, acc_sp.at[pl.ds(row0 + k * 128, 128)])
    plsc.subcore_barrier()

    pltpu.sync_copy(dst_hbm.at[wid], didx_v)

    def _body(ci, _):
        pltpu.sync_copy(ones_v, acc_sp.at[didx_v.at[ci]], add=True)
        return 0
    lax.fori_loop(0, CPW, _body, 0)
    plsc.subcore_barrier()

    for k in range(RPT // 128):
        pltpu.sync_copy(acc_sp.at[pl.ds(row0 + k * 128, 128)], stage_v)
        pltpu.sync_copy(stage_v, deg_out.at[c, pl.ds(row0 + k * 128, 128)])


def _sc_propagate(hs_hbm, src_hbm, dst_hbm, p_out,
                  sidx_v, didx_v, rows_v, acc_sp, sem0):
    c = lax.axis_index("c")
    s = lax.axis_index("s")
    wid = c * NS + s
    row0 = s * RPT

    def _zero(i, _):
        for j in range(F // LANES):
            rows_v[0, i, pl.ds(j * LANES, LANES)] = (
                jnp.zeros((LANES,), jnp.float32))
        return 0
    lax.fori_loop(0, CHUNK, _zero, 0)
    for k in range(RPT // CHUNK):
        pltpu.sync_copy(rows_v.at[0],
                        acc_sp.at[pl.ds(row0 + k * CHUNK, CHUNK)])

    pltpu.sync_copy(src_hbm.at[wid], sidx_v)
    pltpu.sync_copy(dst_hbm.at[wid], didx_v)
    plsc.subcore_barrier()

    def _body(ci, _):
        pltpu.async_copy(hs_hbm.at[sidx_v.at[ci]], rows_v.at[0], sem0).wait()
        pltpu.sync_copy(rows_v.at[0], acc_sp.at[didx_v.at[ci]], add=True)
        return 0
    lax.fori_loop(0, CPW, _body, 0)
    plsc.subcore_barrier()

    for k in range(RPT // CHUNK):
        pltpu.sync_copy(acc_sp.at[pl.ds(row0 + k * CHUNK, CHUNK)],
                        rows_v.at[0])
        pltpu.sync_copy(rows_v.at[0],
                        p_out.at[c, pl.ds(row0 + k * CHUNK, CHUNK)])


_deg_call = functools.partial(
    pl.kernel, _sc_degree,
    out_type=jax.ShapeDtypeStruct((NC, N_PAD, DEGW), jnp.float32),
    mesh=plsc.VectorSubcoreMesh(**_MESH),
    scratch_types=[
        pltpu.VMEM((CPW, CHUNK), jnp.int32),
        pltpu.VMEM((CHUNK, DEGW), jnp.float32),
        pltpu.VMEM((128, DEGW), jnp.float32),
        pltpu.VMEM_SHARED((N_PAD, DEGW), jnp.float32),
    ],
)()

_prop_call = functools.partial(
    pl.kernel, _sc_propagate,
    out_type=jax.ShapeDtypeStruct((NC, N_PAD, F), jnp.float32),
    mesh=plsc.VectorSubcoreMesh(**_MESH),
    scratch_types=[
        pltpu.VMEM((CPW, CHUNK), jnp.int32),
        pltpu.VMEM((CPW, CHUNK), jnp.int32),
        pltpu.VMEM((1, CHUNK, F), jnp.float32),
        pltpu.VMEM_SHARED((N_PAD, F), jnp.float32),
        pltpu.SemaphoreType.DMA,
    ],
)()


RB = 1024
GRID = N_PAD // RB


def _tc_prelude(x_ref, w_ref, b_ref, d0_ref, d1_ref,
                h_ref, hs_ref, dis_ref, invd_ref):
    deg = d0_ref[...][:, :1] + d1_ref[...][:, :1] + 1.0
    dis = lax.rsqrt(deg)
    h = jnp.dot(x_ref[...], w_ref[...], preferred_element_type=jnp.float32)
    h = jnp.maximum(h + b_ref[...], 0.0)
    h_ref[...] = h
    hs_ref[...] = h * dis
    dis_ref[...] = dis
    invd_ref[...] = 1.0 / deg


def _tc_layer(p0_ref, p1_ref, h_ref, x0_ref, dis_ref, invd_ref, w_ref, bs_ref,
              hn_ref, hsn_ref):
    h = h_ref[...]
    dis = dis_ref[...]
    hi = (p0_ref[...] + p1_ref[...]) * dis + h * invd_ref[...]
    t = MIX * hi + RW * h + BETA * x0_ref[...]
    hn = jnp.dot(t, w_ref[...], preferred_element_type=jnp.float32)
    bs = bs_ref[...]
    hn = jnp.maximum(hn - bs, 0.0) + bs
    hn_ref[...] = hn
    hsn_ref[...] = hn * dis


def _tc_output(h_ref, w_ref, b_ref, o_ref):
    o_ref[...] = jnp.dot(h_ref[...], w_ref[...],
                         preferred_element_type=jnp.float32) + b_ref[...]


def _rowspec(cols):
    return pl.BlockSpec((RB, cols), lambda i: (i, 0))


def _fullspec(r, cols):
    return pl.BlockSpec((r, cols), lambda i: (0, 0))


_prelude_call = pl.pallas_call(
    _tc_prelude,
    grid=(GRID,),
    in_specs=[_rowspec(F), _fullspec(F, F), _fullspec(1, F),
              _rowspec(DEGW), _rowspec(DEGW)],
    out_specs=[_rowspec(F), _rowspec(F), _rowspec(1), _rowspec(1)],
    out_shape=[jax.ShapeDtypeStruct((N_PAD, F), jnp.float32),
               jax.ShapeDtypeStruct((N_PAD, F), jnp.float32),
               jax.ShapeDtypeStruct((N_PAD, 1), jnp.float32),
               jax.ShapeDtypeStruct((N_PAD, 1), jnp.float32)],
)

_layer_call = pl.pallas_call(
    _tc_layer,
    grid=(GRID,),
    in_specs=[_rowspec(F), _rowspec(F), _rowspec(F), _rowspec(F),
              _rowspec(1), _rowspec(1), _fullspec(F, F), _fullspec(1, F)],
    out_specs=[_rowspec(F), _rowspec(F)],
    out_shape=[jax.ShapeDtypeStruct((N_PAD, F), jnp.float32),
               jax.ShapeDtypeStruct((N_PAD, F), jnp.float32)],
)

_output_call = pl.pallas_call(
    _tc_output,
    grid=(GRID,),
    in_specs=[_rowspec(F), _fullspec(F, F), _fullspec(1, F)],
    out_specs=_rowspec(F),
    out_shape=jax.ShapeDtypeStruct((N_PAD, F), jnp.float32),
)


def kernel(x, edge_index, W_in, b_in, W_gcn, srelu_bias, W_out, b_out):
    src = edge_index[0]
    dst = edge_index[1]
    npad = E_PAD - E
    src_p = jnp.concatenate([src, jnp.zeros((npad,), jnp.int32)])
    dst_p = jnp.concatenate(
        [dst, N + (jnp.arange(npad, dtype=jnp.int32) % (N_PAD - N))])
    src_r = src_p.reshape(NC * NS, CPW, CHUNK)
    dst_r = dst_p.reshape(NC * NS, CPW, CHUNK)

    x_p = jnp.pad(x, ((0, N_PAD - N), (0, 0)))
    Wo = jnp.pad(W_out, ((0, 0), (0, F - NCLS)))
    bo = jnp.pad(b_out, (0, F - NCLS))

    deg = _deg_call(dst_r)
    d0 = deg[0]
    d1 = deg[1]

    h, hs, dis, invd = _prelude_call(x_p, W_in, b_in.reshape(1, F), d0, d1)
    x0 = h
    for i in range(NL):
        P = _prop_call(hs, src_r, dst_r)
        h, hs = _layer_call(P[0], P[1], h, x0, dis, invd,
                            W_gcn[i], srelu_bias[i].reshape(1, F))
    out = _output_call(h, Wo, bo.reshape(1, F))
    return out[:N, :NCLS]
